# baseline (speedup 1.0000x reference)
import jax
import jax.numpy as jnp
from jax import lax
from jax.experimental import pallas as pl
from jax.experimental.pallas import tpu as pltpu
from jax.experimental.pallas import tpu_sc as plsc

B, L_e, L_c = 1024, 50, 20
V, D = 1000000, 64
PAIRS = B * L_e

_info = plsc.get_sparse_core_info()
NC, NS, L = _info.num_cores, _info.num_subcores, _info.num_lanes
NW = NC * NS
PAIRS_PER_W = PAIRS // NW

CP = 4
SUP = 80
SUBS = SUP // CP
NSUP = PAIRS_PER_W // SUP
NB = 4
NVR = CP * L_c // L


def _q(r):
    return (r & -512) | ((r & 255) << 1) | ((r >> 8) & 1)


def _sc_kernel(ent_ids, ctx_ids, msk, table, ent_out, pool_out,
               ids2, msk2, eids2, eq_v, gidx, rows, erows2, obuf,
               sem_i, sem_m, sem_e, sem_r, sem_o, sem_eg, sem_ef):
    wid = lax.axis_index("s") * NC + lax.axis_index("c")
    base = wid * PAIRS_PER_W
    lane = lax.iota(jnp.int32, L)
    gids = []
    for j in range(NVR):
        pos = lane + j * L
        gid = ((pos >= L_c).astype(jnp.int32)
               + (pos >= 2 * L_c).astype(jnp.int32)
               + (pos >= 3 * L_c).astype(jnp.int32))
        gids.append(gid)

    pltpu.async_copy(ctx_ids.at[pl.ds(base * L_c, SUP * L_c)], ids2.at[0],
                     sem_i.at[0])
    pltpu.async_copy(msk.at[pl.ds(base * L_c, SUP * L_c)], msk2.at[0],
                     sem_m.at[0])
    pltpu.async_copy(ent_ids.at[pl.ds(base, SUP)], eids2.at[0], sem_e.at[0])

    def sup_body(s, carry):
        sl = s & 1
        p0 = base + s * SUP
        pltpu.make_async_copy(ctx_ids.at[pl.ds(p0 * L_c, SUP * L_c)],
                              ids2.at[sl], sem_i.at[sl]).wait()
        pltpu.make_async_copy(msk.at[pl.ds(p0 * L_c, SUP * L_c)],
                              msk2.at[sl], sem_m.at[sl]).wait()
        pltpu.make_async_copy(ent_ids.at[pl.ds(p0, SUP)],
                              eids2.at[sl], sem_e.at[sl]).wait()

        @pl.when(s + 1 < NSUP)
        def _():
            nsl = (s + 1) & 1
            np0 = base + (s + 1) * SUP
            pltpu.async_copy(ctx_ids.at[pl.ds(np0 * L_c, SUP * L_c)],
                             ids2.at[nsl], sem_i.at[nsl])
            pltpu.async_copy(msk.at[pl.ds(np0 * L_c, SUP * L_c)],
                             msk2.at[nsl], sem_m.at[nsl])
            pltpu.async_copy(ent_ids.at[pl.ds(np0, SUP)],
                             eids2.at[nsl], sem_e.at[nsl])

        @pl.when(s >= 2)
        def _():
            poff = base + (s - 2) * SUP
            pltpu.make_async_copy(erows2.at[sl],
                                  ent_out.at[pl.ds(poff, SUP)],
                                  sem_ef.at[sl]).wait()

        for j in range(SUP // L):
            eq_v[pl.ds(j * L, L)] = _q(eids2[sl, pl.ds(j * L, L)])
        pltpu.async_copy(table.at[eq_v], erows2.at[sl], sem_eg.at[sl])

        for c in range(SUBS):
            for j in range(NVR):
                q = c * NVR + j
                ids = ids2[sl, pl.ds(q * L, L)]
                m = msk2[sl, pl.ds(q * L, L)]
                evec = plsc.load_gather(
                    eids2, [jnp.full((L,), sl, jnp.int32), gids[j] + c * CP])
                gidx[c, pl.ds(j * L, L)] = _q(jnp.where(m == 0, ids, evec))

        @pl.when(s >= 2)
        def _():
            poff = base + (s - 2) * SUP
            pltpu.make_async_copy(obuf.at[sl], pool_out.at[pl.ds(poff, SUP)],
                                  sem_o.at[sl]).wait()

        pltpu.make_async_copy(table.at[eq_v], erows2.at[sl],
                              sem_eg.at[sl]).wait()
        pltpu.async_copy(erows2.at[sl], ent_out.at[pl.ds(p0, SUP)],
                         sem_ef.at[sl])

        for j in range(NB):
            pltpu.async_copy(table.at[gidx.at[j]], rows.at[j], sem_r.at[j])

        def sub_body(j, carry2):
            r = j & (NB - 1)
            pltpu.make_async_copy(table.at[gidx.at[j]], rows.at[r],
                                  sem_r.at[r]).wait()
            for p in range(CP):
                cnt = jnp.zeros((L,), jnp.int32)
                for q in range(NVR):
                    pos = lane + q * L
                    m = msk2[sl, pl.ds(j * (CP * L_c) + q * L, L)]
                    sel = (pos >= p * L_c) & (pos < (p + 1) * L_c) & (m == 0)
                    cnt = cnt + plsc.all_reduce_population_count(sel)
                cnt_f = cnt.astype(jnp.float32)
                nmiss = jnp.float32(L_c) - cnt_f
                inv = 1.0 / jnp.maximum(cnt_f, 1.0)
                for q in range(D // L):
                    acc = rows[r, p * L_c, pl.ds(q * L, L)]
                    for k in range(1, L_c):
                        acc = acc + rows[r, p * L_c + k, pl.ds(q * L, L)]
                    ent = erows2[sl, j * CP + p, pl.ds(q * L, L)]
                    obuf[sl, j * CP + p, pl.ds(q * L, L)] = (
                        (acc - nmiss * ent) * inv)

            @pl.when(j + NB < SUBS)
            def _():
                pltpu.async_copy(table.at[gidx.at[j + NB]], rows.at[r],
                                 sem_r.at[r])

            return carry2

        lax.fori_loop(0, SUBS, sub_body, 0)
        pltpu.async_copy(obuf.at[sl], pool_out.at[pl.ds(p0, SUP)], sem_o.at[sl])
        return carry

    lax.fori_loop(0, NSUP, sup_body, 0)
    for s in (NSUP - 2, NSUP - 1):
        poff = base + s * SUP
        pltpu.make_async_copy(obuf.at[s & 1], pool_out.at[pl.ds(poff, SUP)],
                              sem_o.at[s & 1]).wait()
        pltpu.make_async_copy(erows2.at[s & 1], ent_out.at[pl.ds(poff, SUP)],
                              sem_ef.at[s & 1]).wait()


_RB = 512


def _tc_transpose_body(src, dst):
    x = src[...]
    t1 = x[:, : _RB // 2].T
    t2 = x[:, _RB // 2:].T
    dst[...] = jnp.concatenate([t1, t2], axis=1)


def _tc_transpose(table_t):
    grid = (V + _RB - 1) // _RB
    return pl.pallas_call(
        _tc_transpose_body,
        grid=(grid,),
        in_specs=[pl.BlockSpec((D, _RB), lambda i: (0, i))],
        out_specs=pl.BlockSpec((_RB // 2, 2 * D), lambda i: (i, 0)),
        out_shape=jax.ShapeDtypeStruct((grid * _RB // 2, 2 * D), jnp.float32),
    )(table_t)


@jax.jit
def kernel(entity_ids, context_ids, context_padding_mask, table):
    ent_flat = entity_ids.reshape(PAIRS)
    ctx_flat = context_ids.reshape(PAIRS * L_c)
    msk_flat = context_padding_mask.reshape(PAIRS * L_c).astype(jnp.int32)
    table_lin = _tc_transpose(table.T)
    table = table_lin.reshape(table_lin.shape[0] * 2, D)

    mesh = plsc.VectorSubcoreMesh(core_axis_name="c", subcore_axis_name="s")
    f = pl.kernel(
        _sc_kernel,
        mesh=mesh,
        out_type=[
            jax.ShapeDtypeStruct((PAIRS, D), jnp.float32),
            jax.ShapeDtypeStruct((PAIRS, D), jnp.float32),
        ],
        scratch_types=[
            pltpu.VMEM((2, SUP * L_c), jnp.int32),
            pltpu.VMEM((2, SUP * L_c), jnp.int32),
            pltpu.VMEM((2, SUP), jnp.int32),
            pltpu.VMEM((SUP,), jnp.int32),
            pltpu.VMEM((SUBS, CP * L_c), jnp.int32),
            pltpu.VMEM((NB, CP * L_c, D), jnp.float32),
            pltpu.VMEM((2, SUP, D), jnp.float32),
            pltpu.VMEM((2, SUP, D), jnp.float32),
            pltpu.SemaphoreType.DMA((2,)),
            pltpu.SemaphoreType.DMA((2,)),
            pltpu.SemaphoreType.DMA((2,)),
            pltpu.SemaphoreType.DMA((NB,)),
            pltpu.SemaphoreType.DMA((2,)),
            pltpu.SemaphoreType.DMA((2,)),
            pltpu.SemaphoreType.DMA((2,)),
        ],
        compiler_params=pltpu.CompilerParams(
            needs_layout_passes=False, use_tc_tiling_on_sc=False),
    )
    ent_out, pool_out = f(ent_flat, ctx_flat, msk_flat, table)
    return ent_out.reshape(B, L_e, D), pool_out.reshape(B, L_e, D)


# speedup vs baseline: 2.2012x; 2.2012x over previous
import jax
import jax.numpy as jnp
from jax import lax
from jax.experimental import pallas as pl
from jax.experimental.pallas import tpu as pltpu
from jax.experimental.pallas import tpu_sc as plsc

B, L_e, L_c = 1024, 50, 20
V, D = 1000000, 64
PAIRS = B * L_e

_info = plsc.get_sparse_core_info()
NC, NS, L = _info.num_cores, _info.num_subcores, _info.num_lanes
NW = NC * NS
PAIRS_PER_W = PAIRS // NW

CP = 4
SUP = 80
SUBS = SUP // CP
NSUP = PAIRS_PER_W // SUP
NB = 4
NVR = CP * L_c // L


_RB = 4096


def _q(r):
    return (r & -_RB) | ((r & (_RB // 2 - 1)) << 1) | ((r >> 11) & 1)


def _sc_kernel(ent_ids, ctx_ids, msk, table, ent_out, pool_out,
               ids2, msk2, eids2, eq_v, gidx, rows, erows2, obuf,
               sem_i, sem_m, sem_e, sem_r, sem_o, sem_eg, sem_ef):
    wid = lax.axis_index("s") * NC + lax.axis_index("c")
    base = wid * PAIRS_PER_W
    lane = lax.iota(jnp.int32, L)
    gids = []
    for j in range(NVR):
        pos = lane + j * L
        gid = ((pos >= L_c).astype(jnp.int32)
               + (pos >= 2 * L_c).astype(jnp.int32)
               + (pos >= 3 * L_c).astype(jnp.int32))
        gids.append(gid)

    pltpu.async_copy(ctx_ids.at[pl.ds(base * L_c, SUP * L_c)], ids2.at[0],
                     sem_i.at[0])
    pltpu.async_copy(msk.at[pl.ds(base * L_c, SUP * L_c)], msk2.at[0],
                     sem_m.at[0])
    pltpu.async_copy(ent_ids.at[pl.ds(base, SUP)], eids2.at[0], sem_e.at[0])

    def sup_body(s, carry):
        sl = s & 1
        p0 = base + s * SUP
        pltpu.make_async_copy(ctx_ids.at[pl.ds(p0 * L_c, SUP * L_c)],
                              ids2.at[sl], sem_i.at[sl]).wait()
        pltpu.make_async_copy(msk.at[pl.ds(p0 * L_c, SUP * L_c)],
                              msk2.at[sl], sem_m.at[sl]).wait()
        pltpu.make_async_copy(ent_ids.at[pl.ds(p0, SUP)],
                              eids2.at[sl], sem_e.at[sl]).wait()

        @pl.when(s + 1 < NSUP)
        def _():
            nsl = (s + 1) & 1
            np0 = base + (s + 1) * SUP
            pltpu.async_copy(ctx_ids.at[pl.ds(np0 * L_c, SUP * L_c)],
                             ids2.at[nsl], sem_i.at[nsl])
            pltpu.async_copy(msk.at[pl.ds(np0 * L_c, SUP * L_c)],
                             msk2.at[nsl], sem_m.at[nsl])
            pltpu.async_copy(ent_ids.at[pl.ds(np0, SUP)],
                             eids2.at[nsl], sem_e.at[nsl])

        @pl.when(s >= 2)
        def _():
            poff = base + (s - 2) * SUP
            pltpu.make_async_copy(erows2.at[sl],
                                  ent_out.at[pl.ds(poff, SUP)],
                                  sem_ef.at[sl]).wait()

        for j in range(SUP // L):
            eq_v[pl.ds(j * L, L)] = _q(eids2[sl, pl.ds(j * L, L)])
        pltpu.async_copy(table.at[eq_v], erows2.at[sl], sem_eg.at[sl])

        for c in range(SUBS):
            for j in range(NVR):
                q = c * NVR + j
                ids = ids2[sl, pl.ds(q * L, L)]
                m = msk2[sl, pl.ds(q * L, L)]
                evec = plsc.load_gather(
                    eids2, [jnp.full((L,), sl, jnp.int32), gids[j] + c * CP])
                gidx[c, pl.ds(j * L, L)] = _q(jnp.where(m == 0, ids, evec))

        @pl.when(s >= 2)
        def _():
            poff = base + (s - 2) * SUP
            pltpu.make_async_copy(obuf.at[sl], pool_out.at[pl.ds(poff, SUP)],
                                  sem_o.at[sl]).wait()

        pltpu.make_async_copy(table.at[eq_v], erows2.at[sl],
                              sem_eg.at[sl]).wait()
        pltpu.async_copy(erows2.at[sl], ent_out.at[pl.ds(p0, SUP)],
                         sem_ef.at[sl])

        for j in range(NB):
            pltpu.async_copy(table.at[gidx.at[j]], rows.at[j], sem_r.at[j])

        def sub_body(j, carry2):
            r = j & (NB - 1)
            pltpu.make_async_copy(table.at[gidx.at[j]], rows.at[r],
                                  sem_r.at[r]).wait()
            for p in range(CP):
                cnt = jnp.zeros((L,), jnp.int32)
                for q in range(NVR):
                    pos = lane + q * L
                    m = msk2[sl, pl.ds(j * (CP * L_c) + q * L, L)]
                    sel = (pos >= p * L_c) & (pos < (p + 1) * L_c) & (m == 0)
                    cnt = cnt + plsc.all_reduce_population_count(sel)
                cnt_f = cnt.astype(jnp.float32)
                nmiss = jnp.float32(L_c) - cnt_f
                inv = 1.0 / jnp.maximum(cnt_f, 1.0)
                for q in range(D // L):
                    acc = rows[r, p * L_c, pl.ds(q * L, L)]
                    for k in range(1, L_c):
                        acc = acc + rows[r, p * L_c + k, pl.ds(q * L, L)]
                    ent = erows2[sl, j * CP + p, pl.ds(q * L, L)]
                    obuf[sl, j * CP + p, pl.ds(q * L, L)] = (
                        (acc - nmiss * ent) * inv)

            @pl.when(j + NB < SUBS)
            def _():
                pltpu.async_copy(table.at[gidx.at[j + NB]], rows.at[r],
                                 sem_r.at[r])

            return carry2

        lax.fori_loop(0, SUBS, sub_body, 0)
        pltpu.async_copy(obuf.at[sl], pool_out.at[pl.ds(p0, SUP)], sem_o.at[sl])
        return carry

    lax.fori_loop(0, NSUP, sup_body, 0)
    for s in (NSUP - 2, NSUP - 1):
        poff = base + s * SUP
        pltpu.make_async_copy(obuf.at[s & 1], pool_out.at[pl.ds(poff, SUP)],
                              sem_o.at[s & 1]).wait()
        pltpu.make_async_copy(erows2.at[s & 1], ent_out.at[pl.ds(poff, SUP)],
                              sem_ef.at[s & 1]).wait()


def _tc_transpose_body(src, dst):
    x = src[...]
    t1 = x[:, : _RB // 2].T
    t2 = x[:, _RB // 2:].T
    dst[...] = jnp.concatenate([t1, t2], axis=1)


def _tc_transpose(table_t):
    grid = (V + _RB - 1) // _RB
    return pl.pallas_call(
        _tc_transpose_body,
        grid=(grid,),
        in_specs=[pl.BlockSpec((D, _RB), lambda i: (0, i))],
        out_specs=pl.BlockSpec((_RB // 2, 2 * D), lambda i: (i, 0)),
        out_shape=jax.ShapeDtypeStruct((grid * _RB // 2, 2 * D), jnp.float32),
    )(table_t)


@jax.jit
def kernel(entity_ids, context_ids, context_padding_mask, table):
    ent_flat = entity_ids.reshape(PAIRS)
    ctx_flat = context_ids.reshape(PAIRS * L_c)
    msk_flat = context_padding_mask.reshape(PAIRS * L_c).astype(jnp.int32)
    table_lin = _tc_transpose(table.T)
    table = table_lin.reshape(table_lin.shape[0] * 2, D)

    mesh = plsc.VectorSubcoreMesh(core_axis_name="c", subcore_axis_name="s")
    f = pl.kernel(
        _sc_kernel,
        mesh=mesh,
        out_type=[
            jax.ShapeDtypeStruct((PAIRS, D), jnp.float32),
            jax.ShapeDtypeStruct((PAIRS, D), jnp.float32),
        ],
        scratch_types=[
            pltpu.VMEM((2, SUP * L_c), jnp.int32),
            pltpu.VMEM((2, SUP * L_c), jnp.int32),
            pltpu.VMEM((2, SUP), jnp.int32),
            pltpu.VMEM((SUP,), jnp.int32),
            pltpu.VMEM((SUBS, CP * L_c), jnp.int32),
            pltpu.VMEM((NB, CP * L_c, D), jnp.float32),
            pltpu.VMEM((2, SUP, D), jnp.float32),
            pltpu.VMEM((2, SUP, D), jnp.float32),
            pltpu.SemaphoreType.DMA((2,)),
            pltpu.SemaphoreType.DMA((2,)),
            pltpu.SemaphoreType.DMA((2,)),
            pltpu.SemaphoreType.DMA((NB,)),
            pltpu.SemaphoreType.DMA((2,)),
            pltpu.SemaphoreType.DMA((2,)),
            pltpu.SemaphoreType.DMA((2,)),
        ],
        compiler_params=pltpu.CompilerParams(
            needs_layout_passes=False, use_tc_tiling_on_sc=False),
    )
    ent_out, pool_out = f(ent_flat, ctx_flat, msk_flat, table)
    return ent_out.reshape(B, L_e, D), pool_out.reshape(B, L_e, D)


# speedup vs baseline: 2.4259x; 1.1021x over previous
import jax
import jax.numpy as jnp
from jax import lax
from jax.experimental import pallas as pl
from jax.experimental.pallas import tpu as pltpu
from jax.experimental.pallas import tpu_sc as plsc

B, L_e, L_c = 1024, 50, 20
V, D = 1000000, 64
PAIRS = B * L_e

_info = plsc.get_sparse_core_info()
NC, NS, L = _info.num_cores, _info.num_subcores, _info.num_lanes
NW = NC * NS
PAIRS_PER_W = PAIRS // NW

CP = 4
SUP = 80
SUBS = SUP // CP
NSUP = PAIRS_PER_W // SUP
NB = 4
NVR = CP * L_c // L


_RB = 8192


def _q(r):
    return (r & -_RB) | ((r & (_RB // 2 - 1)) << 1) | ((r >> 12) & 1)


def _sc_kernel(ent_ids, ctx_ids, msk, table, ent_out, pool_out,
               ids2, msk2, eids2, eq_v, gidx, rows, erows2, obuf,
               sem_i, sem_m, sem_e, sem_r, sem_o, sem_eg, sem_ef):
    wid = lax.axis_index("s") * NC + lax.axis_index("c")
    base = wid * PAIRS_PER_W
    lane = lax.iota(jnp.int32, L)
    gids = []
    for j in range(NVR):
        pos = lane + j * L
        gid = ((pos >= L_c).astype(jnp.int32)
               + (pos >= 2 * L_c).astype(jnp.int32)
               + (pos >= 3 * L_c).astype(jnp.int32))
        gids.append(gid)

    pltpu.async_copy(ctx_ids.at[pl.ds(base * L_c, SUP * L_c)], ids2.at[0],
                     sem_i.at[0])
    pltpu.async_copy(msk.at[pl.ds(base * L_c, SUP * L_c)], msk2.at[0],
                     sem_m.at[0])
    pltpu.async_copy(ent_ids.at[pl.ds(base, SUP)], eids2.at[0], sem_e.at[0])

    def sup_body(s, carry):
        sl = s & 1
        p0 = base + s * SUP
        pltpu.make_async_copy(ctx_ids.at[pl.ds(p0 * L_c, SUP * L_c)],
                              ids2.at[sl], sem_i.at[sl]).wait()
        pltpu.make_async_copy(msk.at[pl.ds(p0 * L_c, SUP * L_c)],
                              msk2.at[sl], sem_m.at[sl]).wait()
        pltpu.make_async_copy(ent_ids.at[pl.ds(p0, SUP)],
                              eids2.at[sl], sem_e.at[sl]).wait()

        @pl.when(s + 1 < NSUP)
        def _():
            nsl = (s + 1) & 1
            np0 = base + (s + 1) * SUP
            pltpu.async_copy(ctx_ids.at[pl.ds(np0 * L_c, SUP * L_c)],
                             ids2.at[nsl], sem_i.at[nsl])
            pltpu.async_copy(msk.at[pl.ds(np0 * L_c, SUP * L_c)],
                             msk2.at[nsl], sem_m.at[nsl])
            pltpu.async_copy(ent_ids.at[pl.ds(np0, SUP)],
                             eids2.at[nsl], sem_e.at[nsl])

        @pl.when(s >= 2)
        def _():
            poff = base + (s - 2) * SUP
            pltpu.make_async_copy(erows2.at[sl],
                                  ent_out.at[pl.ds(poff, SUP)],
                                  sem_ef.at[sl]).wait()

        for j in range(SUP // L):
            eq_v[pl.ds(j * L, L)] = _q(eids2[sl, pl.ds(j * L, L)])
        pltpu.async_copy(table.at[eq_v], erows2.at[sl], sem_eg.at[sl])

        for c in range(SUBS):
            for j in range(NVR):
                q = c * NVR + j
                ids = ids2[sl, pl.ds(q * L, L)]
                m = msk2[sl, pl.ds(q * L, L)]
                evec = plsc.load_gather(
                    eids2, [jnp.full((L,), sl, jnp.int32), gids[j] + c * CP])
                gidx[c, pl.ds(j * L, L)] = _q(jnp.where(m == 0, ids, evec))

        @pl.when(s >= 2)
        def _():
            poff = base + (s - 2) * SUP
            pltpu.make_async_copy(obuf.at[sl], pool_out.at[pl.ds(poff, SUP)],
                                  sem_o.at[sl]).wait()

        pltpu.make_async_copy(table.at[eq_v], erows2.at[sl],
                              sem_eg.at[sl]).wait()
        pltpu.async_copy(erows2.at[sl], ent_out.at[pl.ds(p0, SUP)],
                         sem_ef.at[sl])

        for j in range(NB):
            pltpu.async_copy(table.at[gidx.at[j]], rows.at[j], sem_r.at[j])

        def sub_body(j, carry2):
            r = j & (NB - 1)
            pltpu.make_async_copy(table.at[gidx.at[j]], rows.at[r],
                                  sem_r.at[r]).wait()
            for p in range(CP):
                cnt = jnp.zeros((L,), jnp.int32)
                for q in range(NVR):
                    pos = lane + q * L
                    m = msk2[sl, pl.ds(j * (CP * L_c) + q * L, L)]
                    sel = (pos >= p * L_c) & (pos < (p + 1) * L_c) & (m == 0)
                    cnt = cnt + plsc.all_reduce_population_count(sel)
                cnt_f = cnt.astype(jnp.float32)
                nmiss = jnp.float32(L_c) - cnt_f
                inv = 1.0 / jnp.maximum(cnt_f, 1.0)
                for q in range(D // L):
                    acc = rows[r, p * L_c, pl.ds(q * L, L)]
                    for k in range(1, L_c):
                        acc = acc + rows[r, p * L_c + k, pl.ds(q * L, L)]
                    ent = erows2[sl, j * CP + p, pl.ds(q * L, L)]
                    obuf[sl, j * CP + p, pl.ds(q * L, L)] = (
                        (acc - nmiss * ent) * inv)

            @pl.when(j + NB < SUBS)
            def _():
                pltpu.async_copy(table.at[gidx.at[j + NB]], rows.at[r],
                                 sem_r.at[r])

            return carry2

        lax.fori_loop(0, SUBS, sub_body, 0)
        pltpu.async_copy(obuf.at[sl], pool_out.at[pl.ds(p0, SUP)], sem_o.at[sl])
        return carry

    lax.fori_loop(0, NSUP, sup_body, 0)
    for s in (NSUP - 2, NSUP - 1):
        poff = base + s * SUP
        pltpu.make_async_copy(obuf.at[s & 1], pool_out.at[pl.ds(poff, SUP)],
                              sem_o.at[s & 1]).wait()
        pltpu.make_async_copy(erows2.at[s & 1], ent_out.at[pl.ds(poff, SUP)],
                              sem_ef.at[s & 1]).wait()


def _tc_transpose_body(src, dst):
    x = src[...]
    t1 = x[:, : _RB // 2].T
    t2 = x[:, _RB // 2:].T
    dst[...] = jnp.concatenate([t1, t2], axis=1)


def _tc_transpose(table_t):
    grid = (V + _RB - 1) // _RB
    return pl.pallas_call(
        _tc_transpose_body,
        grid=(grid,),
        in_specs=[pl.BlockSpec((D, _RB), lambda i: (0, i))],
        out_specs=pl.BlockSpec((_RB // 2, 2 * D), lambda i: (i, 0)),
        out_shape=jax.ShapeDtypeStruct((grid * _RB // 2, 2 * D), jnp.float32),
    )(table_t)


@jax.jit
def kernel(entity_ids, context_ids, context_padding_mask, table):
    ent_flat = entity_ids.reshape(PAIRS)
    ctx_flat = context_ids.reshape(PAIRS * L_c)
    msk_flat = context_padding_mask.reshape(PAIRS * L_c).astype(jnp.int32)
    table_lin = _tc_transpose(table.T)
    table = table_lin.reshape(table_lin.shape[0] * 2, D)

    mesh = plsc.VectorSubcoreMesh(core_axis_name="c", subcore_axis_name="s")
    f = pl.kernel(
        _sc_kernel,
        mesh=mesh,
        out_type=[
            jax.ShapeDtypeStruct((PAIRS, D), jnp.float32),
            jax.ShapeDtypeStruct((PAIRS, D), jnp.float32),
        ],
        scratch_types=[
            pltpu.VMEM((2, SUP * L_c), jnp.int32),
            pltpu.VMEM((2, SUP * L_c), jnp.int32),
            pltpu.VMEM((2, SUP), jnp.int32),
            pltpu.VMEM((SUP,), jnp.int32),
            pltpu.VMEM((SUBS, CP * L_c), jnp.int32),
            pltpu.VMEM((NB, CP * L_c, D), jnp.float32),
            pltpu.VMEM((2, SUP, D), jnp.float32),
            pltpu.VMEM((2, SUP, D), jnp.float32),
            pltpu.SemaphoreType.DMA((2,)),
            pltpu.SemaphoreType.DMA((2,)),
            pltpu.SemaphoreType.DMA((2,)),
            pltpu.SemaphoreType.DMA((NB,)),
            pltpu.SemaphoreType.DMA((2,)),
            pltpu.SemaphoreType.DMA((2,)),
            pltpu.SemaphoreType.DMA((2,)),
        ],
        compiler_params=pltpu.CompilerParams(
            needs_layout_passes=False, use_tc_tiling_on_sc=False),
    )
    ent_out, pool_out = f(ent_flat, ctx_flat, msk_flat, table)
    return ent_out.reshape(B, L_e, D), pool_out.reshape(B, L_e, D)


# speedup vs baseline: 2.5487x; 1.0506x over previous
import jax
import jax.numpy as jnp
from jax import lax
from jax.experimental import pallas as pl
from jax.experimental.pallas import tpu as pltpu
from jax.experimental.pallas import tpu_sc as plsc

B, L_e, L_c = 1024, 50, 20
V, D = 1000000, 64
PAIRS = B * L_e

_info = plsc.get_sparse_core_info()
NC, NS, L = _info.num_cores, _info.num_subcores, _info.num_lanes
NW = NC * NS
PAIRS_PER_W = PAIRS // NW

CP = 4
SUP = 80
SUBS = SUP // CP
NSUP = PAIRS_PER_W // SUP
NB = 4
NVR = CP * L_c // L


_RB = 16384


def _q(r):
    return (r & -_RB) | ((r & (_RB // 2 - 1)) << 1) | ((r >> 13) & 1)


def _sc_kernel(ent_ids, ctx_ids, msk, table, ent_out, pool_out,
               ids2, msk2, eids2, eq_v, gidx, rows, erows2, obuf,
               sem_i, sem_m, sem_e, sem_r, sem_o, sem_eg, sem_ef):
    wid = lax.axis_index("s") * NC + lax.axis_index("c")
    base = wid * PAIRS_PER_W
    lane = lax.iota(jnp.int32, L)
    gids = []
    for j in range(NVR):
        pos = lane + j * L
        gid = ((pos >= L_c).astype(jnp.int32)
               + (pos >= 2 * L_c).astype(jnp.int32)
               + (pos >= 3 * L_c).astype(jnp.int32))
        gids.append(gid)

    pltpu.async_copy(ctx_ids.at[pl.ds(base * L_c, SUP * L_c)], ids2.at[0],
                     sem_i.at[0])
    pltpu.async_copy(msk.at[pl.ds(base * L_c, SUP * L_c)], msk2.at[0],
                     sem_m.at[0])
    pltpu.async_copy(ent_ids.at[pl.ds(base, SUP)], eids2.at[0], sem_e.at[0])

    def sup_body(s, carry):
        sl = s & 1
        p0 = base + s * SUP
        pltpu.make_async_copy(ctx_ids.at[pl.ds(p0 * L_c, SUP * L_c)],
                              ids2.at[sl], sem_i.at[sl]).wait()
        pltpu.make_async_copy(msk.at[pl.ds(p0 * L_c, SUP * L_c)],
                              msk2.at[sl], sem_m.at[sl]).wait()
        pltpu.make_async_copy(ent_ids.at[pl.ds(p0, SUP)],
                              eids2.at[sl], sem_e.at[sl]).wait()

        @pl.when(s + 1 < NSUP)
        def _():
            nsl = (s + 1) & 1
            np0 = base + (s + 1) * SUP
            pltpu.async_copy(ctx_ids.at[pl.ds(np0 * L_c, SUP * L_c)],
                             ids2.at[nsl], sem_i.at[nsl])
            pltpu.async_copy(msk.at[pl.ds(np0 * L_c, SUP * L_c)],
                             msk2.at[nsl], sem_m.at[nsl])
            pltpu.async_copy(ent_ids.at[pl.ds(np0, SUP)],
                             eids2.at[nsl], sem_e.at[nsl])

        @pl.when(s >= 2)
        def _():
            poff = base + (s - 2) * SUP
            pltpu.make_async_copy(erows2.at[sl],
                                  ent_out.at[pl.ds(poff, SUP)],
                                  sem_ef.at[sl]).wait()

        for j in range(SUP // L):
            eq_v[pl.ds(j * L, L)] = _q(eids2[sl, pl.ds(j * L, L)])
        pltpu.async_copy(table.at[eq_v], erows2.at[sl], sem_eg.at[sl])

        for c in range(SUBS):
            for j in range(NVR):
                q = c * NVR + j
                ids = ids2[sl, pl.ds(q * L, L)]
                m = msk2[sl, pl.ds(q * L, L)]
                evec = plsc.load_gather(
                    eids2, [jnp.full((L,), sl, jnp.int32), gids[j] + c * CP])
                gidx[c, pl.ds(j * L, L)] = _q(jnp.where(m == 0, ids, evec))

        @pl.when(s >= 2)
        def _():
            poff = base + (s - 2) * SUP
            pltpu.make_async_copy(obuf.at[sl], pool_out.at[pl.ds(poff, SUP)],
                                  sem_o.at[sl]).wait()

        pltpu.make_async_copy(table.at[eq_v], erows2.at[sl],
                              sem_eg.at[sl]).wait()
        pltpu.async_copy(erows2.at[sl], ent_out.at[pl.ds(p0, SUP)],
                         sem_ef.at[sl])

        for j in range(NB):
            pltpu.async_copy(table.at[gidx.at[j]], rows.at[j], sem_r.at[j])

        def sub_body(j, carry2):
            r = j & (NB - 1)
            pltpu.make_async_copy(table.at[gidx.at[j]], rows.at[r],
                                  sem_r.at[r]).wait()
            for p in range(CP):
                cnt = jnp.zeros((L,), jnp.int32)
                for q in range(NVR):
                    pos = lane + q * L
                    m = msk2[sl, pl.ds(j * (CP * L_c) + q * L, L)]
                    sel = (pos >= p * L_c) & (pos < (p + 1) * L_c) & (m == 0)
                    cnt = cnt + plsc.all_reduce_population_count(sel)
                cnt_f = cnt.astype(jnp.float32)
                nmiss = jnp.float32(L_c) - cnt_f
                inv = 1.0 / jnp.maximum(cnt_f, 1.0)
                for q in range(D // L):
                    acc = rows[r, p * L_c, pl.ds(q * L, L)]
                    for k in range(1, L_c):
                        acc = acc + rows[r, p * L_c + k, pl.ds(q * L, L)]
                    ent = erows2[sl, j * CP + p, pl.ds(q * L, L)]
                    obuf[sl, j * CP + p, pl.ds(q * L, L)] = (
                        (acc - nmiss * ent) * inv)

            @pl.when(j + NB < SUBS)
            def _():
                pltpu.async_copy(table.at[gidx.at[j + NB]], rows.at[r],
                                 sem_r.at[r])

            return carry2

        lax.fori_loop(0, SUBS, sub_body, 0)
        pltpu.async_copy(obuf.at[sl], pool_out.at[pl.ds(p0, SUP)], sem_o.at[sl])
        return carry

    lax.fori_loop(0, NSUP, sup_body, 0)
    for s in (NSUP - 2, NSUP - 1):
        poff = base + s * SUP
        pltpu.make_async_copy(obuf.at[s & 1], pool_out.at[pl.ds(poff, SUP)],
                              sem_o.at[s & 1]).wait()
        pltpu.make_async_copy(erows2.at[s & 1], ent_out.at[pl.ds(poff, SUP)],
                              sem_ef.at[s & 1]).wait()


def _tc_transpose_body(src, dst):
    x = src[...]
    t1 = x[:, : _RB // 2].T
    t2 = x[:, _RB // 2:].T
    dst[...] = jnp.concatenate([t1, t2], axis=1)


def _tc_transpose(table_t):
    grid = (V + _RB - 1) // _RB
    return pl.pallas_call(
        _tc_transpose_body,
        grid=(grid,),
        in_specs=[pl.BlockSpec((D, _RB), lambda i: (0, i))],
        out_specs=pl.BlockSpec((_RB // 2, 2 * D), lambda i: (i, 0)),
        out_shape=jax.ShapeDtypeStruct((grid * _RB // 2, 2 * D), jnp.float32),
    )(table_t)


@jax.jit
def kernel(entity_ids, context_ids, context_padding_mask, table):
    ent_flat = entity_ids.reshape(PAIRS)
    ctx_flat = context_ids.reshape(PAIRS * L_c)
    msk_flat = context_padding_mask.reshape(PAIRS * L_c).astype(jnp.int32)
    table_lin = _tc_transpose(table.T)
    table = table_lin.reshape(table_lin.shape[0] * 2, D)

    mesh = plsc.VectorSubcoreMesh(core_axis_name="c", subcore_axis_name="s")
    f = pl.kernel(
        _sc_kernel,
        mesh=mesh,
        out_type=[
            jax.ShapeDtypeStruct((PAIRS, D), jnp.float32),
            jax.ShapeDtypeStruct((PAIRS, D), jnp.float32),
        ],
        scratch_types=[
            pltpu.VMEM((2, SUP * L_c), jnp.int32),
            pltpu.VMEM((2, SUP * L_c), jnp.int32),
            pltpu.VMEM((2, SUP), jnp.int32),
            pltpu.VMEM((SUP,), jnp.int32),
            pltpu.VMEM((SUBS, CP * L_c), jnp.int32),
            pltpu.VMEM((NB, CP * L_c, D), jnp.float32),
            pltpu.VMEM((2, SUP, D), jnp.float32),
            pltpu.VMEM((2, SUP, D), jnp.float32),
            pltpu.SemaphoreType.DMA((2,)),
            pltpu.SemaphoreType.DMA((2,)),
            pltpu.SemaphoreType.DMA((2,)),
            pltpu.SemaphoreType.DMA((NB,)),
            pltpu.SemaphoreType.DMA((2,)),
            pltpu.SemaphoreType.DMA((2,)),
            pltpu.SemaphoreType.DMA((2,)),
        ],
        compiler_params=pltpu.CompilerParams(
            needs_layout_passes=False, use_tc_tiling_on_sc=False),
    )
    ent_out, pool_out = f(ent_flat, ctx_flat, msk_flat, table)
    return ent_out.reshape(B, L_e, D), pool_out.reshape(B, L_e, D)


# speedup vs baseline: 2.6089x; 1.0236x over previous
import jax
import jax.numpy as jnp
from jax import lax
from jax.experimental import pallas as pl
from jax.experimental.pallas import tpu as pltpu
from jax.experimental.pallas import tpu_sc as plsc

B, L_e, L_c = 1024, 50, 20
V, D = 1000000, 64
PAIRS = B * L_e

_info = plsc.get_sparse_core_info()
NC, NS, L = _info.num_cores, _info.num_subcores, _info.num_lanes
NW = NC * NS
PAIRS_PER_W = PAIRS // NW

CP = 4
SUP = 80
SUBS = SUP // CP
NSUP = PAIRS_PER_W // SUP
NB = 4
NVR = CP * L_c // L


_RB = 32768


def _q(r):
    return (r & -_RB) | ((r & (_RB // 2 - 1)) << 1) | ((r >> 14) & 1)


def _sc_kernel(ent_ids, ctx_ids, msk, table, ent_out, pool_out,
               ids2, msk2, eids2, eq_v, gidx, rows, erows2, obuf,
               sem_i, sem_m, sem_e, sem_r, sem_o, sem_eg, sem_ef):
    wid = lax.axis_index("s") * NC + lax.axis_index("c")
    base = wid * PAIRS_PER_W
    lane = lax.iota(jnp.int32, L)
    gids = []
    for j in range(NVR):
        pos = lane + j * L
        gid = ((pos >= L_c).astype(jnp.int32)
               + (pos >= 2 * L_c).astype(jnp.int32)
               + (pos >= 3 * L_c).astype(jnp.int32))
        gids.append(gid)

    pltpu.async_copy(ctx_ids.at[pl.ds(base * L_c, SUP * L_c)], ids2.at[0],
                     sem_i.at[0])
    pltpu.async_copy(msk.at[pl.ds(base * L_c, SUP * L_c)], msk2.at[0],
                     sem_m.at[0])
    pltpu.async_copy(ent_ids.at[pl.ds(base, SUP)], eids2.at[0], sem_e.at[0])

    def sup_body(s, carry):
        sl = s & 1
        p0 = base + s * SUP
        pltpu.make_async_copy(ctx_ids.at[pl.ds(p0 * L_c, SUP * L_c)],
                              ids2.at[sl], sem_i.at[sl]).wait()
        pltpu.make_async_copy(msk.at[pl.ds(p0 * L_c, SUP * L_c)],
                              msk2.at[sl], sem_m.at[sl]).wait()
        pltpu.make_async_copy(ent_ids.at[pl.ds(p0, SUP)],
                              eids2.at[sl], sem_e.at[sl]).wait()

        @pl.when(s + 1 < NSUP)
        def _():
            nsl = (s + 1) & 1
            np0 = base + (s + 1) * SUP
            pltpu.async_copy(ctx_ids.at[pl.ds(np0 * L_c, SUP * L_c)],
                             ids2.at[nsl], sem_i.at[nsl])
            pltpu.async_copy(msk.at[pl.ds(np0 * L_c, SUP * L_c)],
                             msk2.at[nsl], sem_m.at[nsl])
            pltpu.async_copy(ent_ids.at[pl.ds(np0, SUP)],
                             eids2.at[nsl], sem_e.at[nsl])

        @pl.when(s >= 2)
        def _():
            poff = base + (s - 2) * SUP
            pltpu.make_async_copy(erows2.at[sl],
                                  ent_out.at[pl.ds(poff, SUP)],
                                  sem_ef.at[sl]).wait()

        for j in range(SUP // L):
            eq_v[pl.ds(j * L, L)] = _q(eids2[sl, pl.ds(j * L, L)])
        pltpu.async_copy(table.at[eq_v], erows2.at[sl], sem_eg.at[sl])

        for c in range(SUBS):
            for j in range(NVR):
                q = c * NVR + j
                ids = ids2[sl, pl.ds(q * L, L)]
                m = msk2[sl, pl.ds(q * L, L)]
                evec = plsc.load_gather(
                    eids2, [jnp.full((L,), sl, jnp.int32), gids[j] + c * CP])
                gidx[c, pl.ds(j * L, L)] = _q(jnp.where(m == 0, ids, evec))

        @pl.when(s >= 2)
        def _():
            poff = base + (s - 2) * SUP
            pltpu.make_async_copy(obuf.at[sl], pool_out.at[pl.ds(poff, SUP)],
                                  sem_o.at[sl]).wait()

        pltpu.make_async_copy(table.at[eq_v], erows2.at[sl],
                              sem_eg.at[sl]).wait()
        pltpu.async_copy(erows2.at[sl], ent_out.at[pl.ds(p0, SUP)],
                         sem_ef.at[sl])

        for j in range(NB):
            pltpu.async_copy(table.at[gidx.at[j]], rows.at[j], sem_r.at[j])

        def sub_body(j, carry2):
            r = j & (NB - 1)
            pltpu.make_async_copy(table.at[gidx.at[j]], rows.at[r],
                                  sem_r.at[r]).wait()
            for p in range(CP):
                cnt = jnp.zeros((L,), jnp.int32)
                for q in range(NVR):
                    pos = lane + q * L
                    m = msk2[sl, pl.ds(j * (CP * L_c) + q * L, L)]
                    sel = (pos >= p * L_c) & (pos < (p + 1) * L_c) & (m == 0)
                    cnt = cnt + plsc.all_reduce_population_count(sel)
                cnt_f = cnt.astype(jnp.float32)
                nmiss = jnp.float32(L_c) - cnt_f
                inv = 1.0 / jnp.maximum(cnt_f, 1.0)
                for q in range(D // L):
                    acc = rows[r, p * L_c, pl.ds(q * L, L)]
                    for k in range(1, L_c):
                        acc = acc + rows[r, p * L_c + k, pl.ds(q * L, L)]
                    ent = erows2[sl, j * CP + p, pl.ds(q * L, L)]
                    obuf[sl, j * CP + p, pl.ds(q * L, L)] = (
                        (acc - nmiss * ent) * inv)

            @pl.when(j + NB < SUBS)
            def _():
                pltpu.async_copy(table.at[gidx.at[j + NB]], rows.at[r],
                                 sem_r.at[r])

            return carry2

        lax.fori_loop(0, SUBS, sub_body, 0)
        pltpu.async_copy(obuf.at[sl], pool_out.at[pl.ds(p0, SUP)], sem_o.at[sl])
        return carry

    lax.fori_loop(0, NSUP, sup_body, 0)
    for s in (NSUP - 2, NSUP - 1):
        poff = base + s * SUP
        pltpu.make_async_copy(obuf.at[s & 1], pool_out.at[pl.ds(poff, SUP)],
                              sem_o.at[s & 1]).wait()
        pltpu.make_async_copy(erows2.at[s & 1], ent_out.at[pl.ds(poff, SUP)],
                              sem_ef.at[s & 1]).wait()


def _tc_transpose_body(src, dst):
    x = src[...]
    t1 = x[:, : _RB // 2].T
    t2 = x[:, _RB // 2:].T
    dst[...] = jnp.concatenate([t1, t2], axis=1)


def _tc_transpose(table_t):
    grid = (V + _RB - 1) // _RB
    return pl.pallas_call(
        _tc_transpose_body,
        grid=(grid,),
        in_specs=[pl.BlockSpec((D, _RB), lambda i: (0, i))],
        out_specs=pl.BlockSpec((_RB // 2, 2 * D), lambda i: (i, 0)),
        out_shape=jax.ShapeDtypeStruct((grid * _RB // 2, 2 * D), jnp.float32),
    )(table_t)


@jax.jit
def kernel(entity_ids, context_ids, context_padding_mask, table):
    ent_flat = entity_ids.reshape(PAIRS)
    ctx_flat = context_ids.reshape(PAIRS * L_c)
    msk_flat = context_padding_mask.reshape(PAIRS * L_c).astype(jnp.int32)
    table_lin = _tc_transpose(table.T)
    table = table_lin.reshape(table_lin.shape[0] * 2, D)

    mesh = plsc.VectorSubcoreMesh(core_axis_name="c", subcore_axis_name="s")
    f = pl.kernel(
        _sc_kernel,
        mesh=mesh,
        out_type=[
            jax.ShapeDtypeStruct((PAIRS, D), jnp.float32),
            jax.ShapeDtypeStruct((PAIRS, D), jnp.float32),
        ],
        scratch_types=[
            pltpu.VMEM((2, SUP * L_c), jnp.int32),
            pltpu.VMEM((2, SUP * L_c), jnp.int32),
            pltpu.VMEM((2, SUP), jnp.int32),
            pltpu.VMEM((SUP,), jnp.int32),
            pltpu.VMEM((SUBS, CP * L_c), jnp.int32),
            pltpu.VMEM((NB, CP * L_c, D), jnp.float32),
            pltpu.VMEM((2, SUP, D), jnp.float32),
            pltpu.VMEM((2, SUP, D), jnp.float32),
            pltpu.SemaphoreType.DMA((2,)),
            pltpu.SemaphoreType.DMA((2,)),
            pltpu.SemaphoreType.DMA((2,)),
            pltpu.SemaphoreType.DMA((NB,)),
            pltpu.SemaphoreType.DMA((2,)),
            pltpu.SemaphoreType.DMA((2,)),
            pltpu.SemaphoreType.DMA((2,)),
        ],
        compiler_params=pltpu.CompilerParams(
            needs_layout_passes=False, use_tc_tiling_on_sc=False),
    )
    ent_out, pool_out = f(ent_flat, ctx_flat, msk_flat, table)
    return ent_out.reshape(B, L_e, D), pool_out.reshape(B, L_e, D)
